# baseline (device time: 363218 ns/iter reference)
import functools

import jax
import jax.numpy as jnp
from jax import lax
from jax.experimental import pallas as pl
from jax.experimental.pallas import tpu as pltpu

N_DEV = 32
HOPS_R = N_DEV // 2
HOPS_L = N_DEV - 1 - HOPS_R


def kernel(x, w_mat):
    m_per, k = x.shape
    _, n_per = w_mat.shape

    xb = x.astype(jnp.bfloat16)
    wb = w_mat.astype(jnp.bfloat16)

    def body(x_ref, w_ref, out_ref, comm_ref,
             send_r_sems, recv_r_sems, send_l_sems, recv_l_sems):
        my = lax.axis_index("i")
        right = lax.rem(my + 1, N_DEV)
        left = lax.rem(my + N_DEV - 1, N_DEV)

        barrier_sem = pltpu.get_barrier_semaphore()
        for nbr in (left, right):
            pl.semaphore_signal(
                barrier_sem, inc=1,
                device_id=(nbr,), device_id_type=pl.DeviceIdType.MESH,
            )
        pl.semaphore_wait(barrier_sem, 2)

        comm_ref[my] = x_ref[...]

        def slot(origin):
            return lax.rem(origin + 4 * N_DEV, N_DEV)

        def mk_send(h, origin, dst, sems_s, sems_r):
            return pltpu.make_async_remote_copy(
                src_ref=comm_ref.at[slot(origin)],
                dst_ref=comm_ref.at[slot(origin)],
                send_sem=sems_s.at[h],
                recv_sem=sems_r.at[h],
                device_id=(dst,),
                device_id_type=pl.DeviceIdType.MESH,
            )

        def mk_recv(h, origin, sems_s, sems_r):
            return pltpu.make_async_remote_copy(
                src_ref=comm_ref.at[slot(origin)],
                dst_ref=comm_ref.at[slot(origin)],
                send_sem=sems_s.at[h],
                recv_sem=sems_r.at[h],
                device_id=(right,),
                device_id_type=pl.DeviceIdType.MESH,
            )

        def compute(origin):
            s = slot(origin)
            y = lax.dot_general(
                comm_ref[s], w_ref[...],
                (((1,), (0,)), ((), ())),
                preferred_element_type=jnp.float32,
            )
            out_ref[pl.ds(s * m_per, m_per), :] = jnp.maximum(y, 0.0)

        sends = []

        s0r = mk_send(0, my, right, send_r_sems, recv_r_sems)
        s0r.start()
        sends.append(s0r)
        s0l = mk_send(0, my, left, send_l_sems, recv_l_sems)
        s0l.start()
        sends.append(s0l)

        compute(my)

        for h in range(HOPS_R):
            org_r = my - h - 1
            mk_recv(h, org_r, send_r_sems, recv_r_sems).wait_recv()
            if h + 1 < HOPS_R:
                s = mk_send(h + 1, org_r, right, send_r_sems, recv_r_sems)
                s.start()
                sends.append(s)
            if h < HOPS_L:
                org_l = my + h + 1
                mk_recv(h, org_l, send_l_sems, recv_l_sems).wait_recv()
                if h + 1 < HOPS_L:
                    s = mk_send(h + 1, org_l, left, send_l_sems, recv_l_sems)
                    s.start()
                    sends.append(s)
                compute(org_l)
            compute(org_r)

        for s in sends:
            s.wait_send()

    return pl.pallas_call(
        body,
        out_shape=jax.ShapeDtypeStruct((N_DEV * m_per, n_per), jnp.float32),
        in_specs=[
            pl.BlockSpec(memory_space=pltpu.VMEM),
            pl.BlockSpec(memory_space=pltpu.VMEM),
        ],
        out_specs=pl.BlockSpec(memory_space=pltpu.VMEM),
        scratch_shapes=[
            pltpu.VMEM((N_DEV, m_per, k), jnp.bfloat16),
            pltpu.SemaphoreType.DMA((HOPS_R,)),
            pltpu.SemaphoreType.DMA((HOPS_R,)),
            pltpu.SemaphoreType.DMA((HOPS_L,)),
            pltpu.SemaphoreType.DMA((HOPS_L,)),
        ],
        compiler_params=pltpu.CompilerParams(collective_id=0),
    )(xb, wb)


# device time: 215984 ns/iter; 1.6817x vs baseline; 1.6817x over previous
import functools

import jax
import jax.numpy as jnp
from jax import lax
from jax.experimental import pallas as pl
from jax.experimental.pallas import tpu as pltpu

N_DEV = 32
HOPS_R = N_DEV // 2
HOPS_L = N_DEV - 1 - HOPS_R


def kernel(x, w_mat):
    m_per, k = x.shape
    _, n_per = w_mat.shape

    xb = x.astype(jnp.bfloat16)
    wb = w_mat.astype(jnp.bfloat16)

    def body(x_ref, w_ref, out_ref, comm_ref,
             send_r_sems, recv_r_sems, send_l_sems, recv_l_sems):
        my = lax.axis_index("i")

        z_m = my // 8
        r_m = lax.rem(my, 8)
        y_m = r_m // 2
        x_m = lax.rem((r_m + 1) // 2, 2)
        s_m = 4 * z_m + jnp.where(lax.rem(z_m, 2) == 0, y_m, 3 - y_m)
        ring_pos = jnp.where(x_m == 0, s_m, 31 - s_m)

        def mesh_of_ring(p):
            p = lax.rem(p + 4 * N_DEV, N_DEV)
            s = jnp.where(p < 16, p, 31 - p)
            x = jnp.where(p < 16, 0, 1)
            z = s // 4
            t = lax.rem(s, 4)
            y = jnp.where(lax.rem(z, 2) == 0, t, 3 - t)
            r = 2 * y + jnp.where(lax.rem(y, 2) == 0, x, 1 - x)
            return 8 * z + r

        right = mesh_of_ring(ring_pos + 1)
        left = mesh_of_ring(ring_pos - 1)

        barrier_sem = pltpu.get_barrier_semaphore()
        for nbr in (left, right):
            pl.semaphore_signal(
                barrier_sem, inc=1,
                device_id=(nbr,), device_id_type=pl.DeviceIdType.MESH,
            )
        pl.semaphore_wait(barrier_sem, 2)

        comm_ref[my] = x_ref[...]

        def slot(origin):
            return lax.rem(origin + 4 * N_DEV, N_DEV)

        def mk_send(h, origin, dst, sems_s, sems_r):
            return pltpu.make_async_remote_copy(
                src_ref=comm_ref.at[slot(origin)],
                dst_ref=comm_ref.at[slot(origin)],
                send_sem=sems_s.at[h],
                recv_sem=sems_r.at[h],
                device_id=(dst,),
                device_id_type=pl.DeviceIdType.MESH,
            )

        def mk_recv(h, origin, sems_s, sems_r):
            return pltpu.make_async_remote_copy(
                src_ref=comm_ref.at[slot(origin)],
                dst_ref=comm_ref.at[slot(origin)],
                send_sem=sems_s.at[h],
                recv_sem=sems_r.at[h],
                device_id=(right,),
                device_id_type=pl.DeviceIdType.MESH,
            )

        def compute(origin):
            s = slot(origin)
            y = lax.dot_general(
                comm_ref[s], w_ref[...],
                (((1,), (0,)), ((), ())),
                preferred_element_type=jnp.float32,
            )
            out_ref[pl.ds(s * m_per, m_per), :] = jnp.maximum(y, 0.0)

        sends = []

        s0r = mk_send(0, my, right, send_r_sems, recv_r_sems)
        s0r.start()
        sends.append(s0r)
        s0l = mk_send(0, my, left, send_l_sems, recv_l_sems)
        s0l.start()
        sends.append(s0l)

        compute(my)

        for h in range(HOPS_R):
            org_r = mesh_of_ring(ring_pos - h - 1)
            mk_recv(h, org_r, send_r_sems, recv_r_sems).wait_recv()
            if h + 1 < HOPS_R:
                s = mk_send(h + 1, org_r, right, send_r_sems, recv_r_sems)
                s.start()
                sends.append(s)
            if h < HOPS_L:
                org_l = mesh_of_ring(ring_pos + h + 1)
                mk_recv(h, org_l, send_l_sems, recv_l_sems).wait_recv()
                if h + 1 < HOPS_L:
                    s = mk_send(h + 1, org_l, left, send_l_sems, recv_l_sems)
                    s.start()
                    sends.append(s)
                compute(org_l)
            compute(org_r)

        for s in sends:
            s.wait_send()

    return pl.pallas_call(
        body,
        out_shape=jax.ShapeDtypeStruct((N_DEV * m_per, n_per), jnp.float32),
        in_specs=[
            pl.BlockSpec(memory_space=pltpu.VMEM),
            pl.BlockSpec(memory_space=pltpu.VMEM),
        ],
        out_specs=pl.BlockSpec(memory_space=pltpu.VMEM),
        scratch_shapes=[
            pltpu.VMEM((N_DEV, m_per, k), jnp.bfloat16),
            pltpu.SemaphoreType.DMA((HOPS_R,)),
            pltpu.SemaphoreType.DMA((HOPS_R,)),
            pltpu.SemaphoreType.DMA((HOPS_L,)),
            pltpu.SemaphoreType.DMA((HOPS_L,)),
        ],
        compiler_params=pltpu.CompilerParams(collective_id=0),
    )(xb, wb)


# device time: 187555 ns/iter; 1.9366x vs baseline; 1.1516x over previous
import jax
import jax.numpy as jnp
from jax import lax
from jax.experimental import pallas as pl
from jax.experimental.pallas import tpu as pltpu

N_DEV = 32
HOPS = N_DEV // 2
N_SUB = 2


def kernel(x, w_mat):
    m_per, k = x.shape
    _, n_per = w_mat.shape
    m_sub = m_per // N_SUB

    xb = x.astype(jnp.bfloat16)
    wb = w_mat.astype(jnp.bfloat16)

    def body(x_ref, w_ref, out_ref, comm_ref,
             send_r_sems, recv_r_sems, send_l_sems, recv_l_sems):
        my = lax.axis_index("i")

        z_m = my // 8
        r_m = lax.rem(my, 8)
        y_m = r_m // 2
        x_m = lax.rem((r_m + 1) // 2, 2)
        s_m = 4 * z_m + jnp.where(lax.rem(z_m, 2) == 0, y_m, 3 - y_m)
        ring_pos = jnp.where(x_m == 0, s_m, 31 - s_m)

        def mesh_of_ring(p):
            p = lax.rem(p + 4 * N_DEV, N_DEV)
            s = jnp.where(p < 16, p, 31 - p)
            x_ = jnp.where(p < 16, 0, 1)
            z = s // 4
            t = lax.rem(s, 4)
            y = jnp.where(lax.rem(z, 2) == 0, t, 3 - t)
            r = 2 * y + jnp.where(lax.rem(y, 2) == 0, x_, 1 - x_)
            return 8 * z + r

        right = mesh_of_ring(ring_pos + 1)
        left = mesh_of_ring(ring_pos - 1)

        barrier_sem = pltpu.get_barrier_semaphore()
        for nbr in (left, right):
            pl.semaphore_signal(
                barrier_sem, inc=1,
                device_id=(nbr,), device_id_type=pl.DeviceIdType.MESH,
            )
        pl.semaphore_wait(barrier_sem, 2)

        comm_ref[my] = x_ref[...]

        sends = []

        def mk(go_right, h, origin, j):
            sems_s, sems_r, dst = (
                (send_r_sems, recv_r_sems, right) if go_right
                else (send_l_sems, recv_l_sems, left)
            )
            return pltpu.make_async_remote_copy(
                src_ref=comm_ref.at[origin, pl.ds(j * m_sub, m_sub)],
                dst_ref=comm_ref.at[origin, pl.ds(j * m_sub, m_sub)],
                send_sem=sems_s.at[h, j],
                recv_sem=sems_r.at[h, j],
                device_id=(dst,),
                device_id_type=pl.DeviceIdType.MESH,
            )

        def start_send(go_right, h, origin, j):
            s = mk(go_right, h, origin, j)
            s.start()
            sends.append(s)

        def compute(origin):
            y = lax.dot_general(
                comm_ref[origin], w_ref[...],
                (((1,), (0,)), ((), ())),
                preferred_element_type=jnp.float32,
            )
            out_ref[pl.ds(origin * m_per, m_per), :] = jnp.maximum(y, 0.0)

        for j in range(N_SUB):
            start_send(True, 0, my, j)
        for j in range(N_SUB):
            start_send(False, 0, my, j)

        compute(my)

        for h in range(HOPS):
            org_r = mesh_of_ring(ring_pos - h - 1)
            org_l = mesh_of_ring(ring_pos + h + 1)
            r_subs = range(N_SUB) if h < HOPS - 1 else (0,)
            l_subs = range(N_SUB) if h < HOPS - 1 else (1,)
            for j in r_subs:
                mk(True, h, org_r, j).wait_recv()
                if h + 1 < HOPS - 1 or (h + 1 == HOPS - 1 and j == 0):
                    start_send(True, h + 1, org_r, j)
            for j in l_subs:
                mk(False, h, org_l, j).wait_recv()
                if h + 1 < HOPS - 1 or (h + 1 == HOPS - 1 and j == 1):
                    start_send(False, h + 1, org_l, j)
            compute(org_r)
            if h < HOPS - 1:
                compute(org_l)

        for s in sends:
            s.wait_send()

    return pl.pallas_call(
        body,
        out_shape=jax.ShapeDtypeStruct((N_DEV * m_per, n_per), jnp.float32),
        in_specs=[
            pl.BlockSpec(memory_space=pltpu.VMEM),
            pl.BlockSpec(memory_space=pltpu.VMEM),
        ],
        out_specs=pl.BlockSpec(memory_space=pltpu.VMEM),
        scratch_shapes=[
            pltpu.VMEM((N_DEV, m_per, k), jnp.bfloat16),
            pltpu.SemaphoreType.DMA((HOPS, N_SUB)),
            pltpu.SemaphoreType.DMA((HOPS, N_SUB)),
            pltpu.SemaphoreType.DMA((HOPS, N_SUB)),
            pltpu.SemaphoreType.DMA((HOPS, N_SUB)),
        ],
        compiler_params=pltpu.CompilerParams(collective_id=0),
    )(xb, wb)
